# baseline (speedup 1.0000x reference)
import functools

import jax
import jax.numpy as jnp
from jax import lax
from jax.experimental import pallas as pl
from jax.experimental.pallas import tpu as pltpu
from jax.experimental.pallas import tpu_sc as plsc

_NC = 2
_NS = 16
_C = 100
_K = 10
_F = 128


@functools.lru_cache(maxsize=None)
def _make_deg(N, E):
    NW = _NC * _NS
    EP = E // NW
    NR = N // 16
    assert E % NW == 0 and EP % 16 == 0 and N % 16 == 0
    mesh = plsc.VectorSubcoreMesh(core_axis_name="c", subcore_axis_name="s")

    @functools.partial(
        pl.kernel,
        out_type=jax.ShapeDtypeStruct((2, NW, NR, 16), jnp.float32),
        mesh=mesh,
        compiler_params=pltpu.CompilerParams(needs_layout_passes=False),
        scratch_types=[
            pltpu.VMEM((EP,), jnp.int32),
            pltpu.VMEM((NR, 16), jnp.float32),
        ],
    )
    def deg(srcv, dstv, out, idxv, hist):
        c = lax.axis_index("c")
        s = lax.axis_index("s")
        w = c * _NS + s
        ones16 = jnp.ones((16,), jnp.float32)

        U = 5
        assert NR % U == 0 and (EP // 16) % U == 0

        def zbody(i, carry):
            for k in range(U):
                hist[i * U + k, :] = jnp.zeros((16,), jnp.float32)
            return carry

        def abody(j, carry):
            for k in range(U):
                iv = idxv[pl.ds((j * U + k) * 16, 16)]
                plsc.addupdate_scatter(
                    hist,
                    [lax.shift_right_logical(iv, 4), lax.bitwise_and(iv, 15)],
                    ones16)
            return carry

        for row, ix_hbm in ((0, srcv), (1, dstv)):
            lax.fori_loop(0, NR // U, zbody, 0)
            pltpu.sync_copy(ix_hbm.at[pl.ds(w * EP, EP)], idxv)
            lax.fori_loop(0, EP // 16 // U, abody, 0)
            pltpu.sync_copy(hist, out.at[row, w])

    return deg


@functools.lru_cache(maxsize=None)
def _make_agg(N, E, T, full=False):
    EP = E // _NS if full else E // (_NC * _NS)
    NCH = EP // _C
    NG = NCH // _K
    assert EP % _C == 0 and NCH % _K == 0 and NG % 2 == 0 and _K % 2 == 0
    SR = 8 * (N // (8 * _NS))
    REM = N - SR * _NS
    ZR = 16
    assert SR % ZR == 0 and REM % ZR == 0
    mesh = plsc.VectorSubcoreMesh(core_axis_name="c", subcore_axis_name="s")

    @functools.partial(
        pl.kernel,
        out_type=jax.ShapeDtypeStruct((_NC, N, _F), jnp.float32),
        mesh=mesh,
        compiler_params=pltpu.CompilerParams(needs_layout_passes=False),
        scratch_types=[
            pltpu.VMEM((_K, _C), jnp.int32),
            pltpu.VMEM((_K, _C), jnp.int32),
            pltpu.VMEM((_K, _C), jnp.int32),
            pltpu.VMEM((_K, _C), jnp.int32),
            pltpu.VMEM((_C, _F), jnp.float32),
            pltpu.VMEM((_C, _F), jnp.float32),
            pltpu.VMEM((_C, _F), jnp.float32),
            pltpu.VMEM((ZR, _F), jnp.float32),
            pltpu.VMEM_SHARED((N, _F), jnp.float32),
            pltpu.SemaphoreType.DMA,
            pltpu.SemaphoreType.DMA,
            pltpu.SemaphoreType.DMA,
            pltpu.SemaphoreType.DMA,
            pltpu.SemaphoreType.DMA,
            pltpu.SemaphoreType.DMA,
            pltpu.SemaphoreType.DMA,
            pltpu.SemaphoreType.DMA,
        ],
    )
    def agg(tbl, srcix, dstix, out, s0, d0, s1, d1, rb0, rb1, rb2, zb, acc,
            is0, is1, gs0, gs1, gs2, ss0, ss1, ss2):
        c = lax.axis_index("c")
        s = lax.axis_index("s")

        def stage(g, sbuf, dbuf, sem):
            pltpu.async_copy(srcix.at[c, s, g], sbuf, sem)
            pltpu.async_copy(dstix.at[c, s, g], dbuf, sem)

        def stage_wait(g, sbuf, dbuf, sem):
            pltpu.make_async_copy(srcix.at[c, s, g], sbuf, sem).wait()
            pltpu.make_async_copy(dstix.at[c, s, g], dbuf, sem).wait()

        rb = (rb0, rb1, rb2)
        gs = (gs0, gs1, gs2)
        ss = (ss0, ss1, ss2)

        def zbody(i, carry):
            r = i // (_F // 16)
            col = (i % (_F // 16)) * 16
            zb[r, pl.ds(col, 16)] = jnp.zeros((16,), jnp.float32)
            return carry

        lax.fori_loop(0, ZR * (_F // 16), zbody, 0)

        zd = [pltpu.async_copy(zb, acc.at[pl.ds(s * SR + i * ZR, ZR)], is0)
              for i in range(SR // ZR)]
        if REM:
            @pl.when(s == 0)
            def _():
                for i in range(REM // ZR):
                    pltpu.sync_copy(zb, acc.at[pl.ds(_NS * SR + i * ZR, ZR)])
        for d in zd:
            d.wait()
        stage(0, s0, d0, is0)
        stage(1, s1, d1, is1)
        plsc.subcore_barrier()

        def gbody(gg, carry):
            g0 = 2 * gg
            g1 = g0 + 1

            def bufs(u):
                return (s0, d0, u) if u < _K else (s1, d1, u - _K)

            stage_wait(g0, s0, d0, is0)
            gd, sd = {}, {}
            for u in range(2 * _K):
                if u == _K:
                    stage_wait(g1, s1, d1, is1)
                b = u % 3
                if u >= 3:
                    sd[u - 3].wait()
                    if u - 3 == _K - 1:
                        @pl.when(gg < NG // 2 - 1)
                        def _():
                            stage(g0 + 2, s0, d0, is0)
                sb, db, r = bufs(u)
                gd[u] = pltpu.async_copy(tbl.at[sb.at[r]], rb[b], gs[b])
                if u >= 1:
                    _, pdb, pr = bufs(u - 1)
                    gd[u - 1].wait()
                    sd[u - 1] = pltpu.async_copy(
                        rb[(u - 1) % 3], acc.at[pdb.at[pr]], ss[(u - 1) % 3],
                        add=True)
            gd[2 * _K - 1].wait()
            _, pdb, pr = bufs(2 * _K - 1)
            sd[2 * _K - 1] = pltpu.async_copy(
                rb[(2 * _K - 1) % 3], acc.at[pdb.at[pr]], ss[(2 * _K - 1) % 3],
                add=True)
            for u in range(2 * _K - 3, 2 * _K):
                sd[u].wait()

            @pl.when(gg < NG // 2 - 1)
            def _():
                stage(g1 + 2, s1, d1, is1)

            return carry

        lax.fori_loop(0, NG // 2, gbody, 0)
        plsc.subcore_barrier()
        pltpu.sync_copy(acc.at[pl.ds(s * SR, SR)], out.at[c, pl.ds(s * SR, SR)])
        if REM:
            @pl.when(s == 0)
            def _():
                pltpu.sync_copy(acc.at[pl.ds(_NS * SR, REM)],
                                out.at[c, pl.ds(_NS * SR, REM)])

    return agg


def _bf16_dot(a, b, trans_b=False):
    dn = (((1,), (1,)), ((), ())) if trans_b else (((1,), (0,)), ((), ()))
    return lax.dot_general(a.astype(jnp.bfloat16), b.astype(jnp.bfloat16),
                           dn, preferred_element_type=jnp.float32)


def _tc_scales(hist):
    def body(h_ref, o_ref):
        d = jnp.sum(h_ref[...], axis=1)
        o_ref[...] = 1.0 / jnp.sqrt(jnp.maximum(d, 1.0))

    return pl.pallas_call(
        body,
        out_shape=jax.ShapeDtypeStruct((2, hist.shape[2]), jnp.float32),
    )(hist)


def _tc_big_mm_relu(Bm, W, mt=256):
    N, K = Bm.shape
    Fo = W.shape[1]
    M = pl.cdiv(N, mt)
    Wb = W.astype(jnp.bfloat16)

    def body(b_ref, w_ref, o_ref):
        o_ref[...] = jnp.maximum(_bf16_dot(b_ref[...], w_ref[...]), 0.0)

    return pl.pallas_call(
        body,
        grid=(M,),
        in_specs=[
            pl.BlockSpec((mt, K), lambda m: (m, 0)),
            pl.BlockSpec((K, Fo), lambda m: (0, 0)),
        ],
        out_specs=pl.BlockSpec((mt, Fo), lambda m: (m, 0)),
        out_shape=jax.ShapeDtypeStruct((N, Fo), jnp.float32),
    )(Bm, Wb)


def _tc_big_out(Xa, Xb, trans_b, mt=1000, nt=2560):
    N = Xa.shape[0]
    N2 = Xb.shape[0] if trans_b else Xb.shape[1]
    K = Xa.shape[1]
    M, NT = pl.cdiv(N, mt), pl.cdiv(N2, nt)

    def body(a_ref, b_ref, o_ref):
        o_ref[...] = jax.nn.sigmoid(_bf16_dot(a_ref[...], b_ref[...], trans_b))

    if trans_b:
        b_spec = pl.BlockSpec((nt, K), lambda m, j: (j, 0))
    else:
        b_spec = pl.BlockSpec((K, nt), lambda m, j: (0, j))
    return pl.pallas_call(
        body,
        grid=(M, NT),
        in_specs=[pl.BlockSpec((mt, K), lambda m, j: (m, 0)), b_spec],
        out_specs=pl.BlockSpec((mt, nt), lambda m, j: (m, j)),
        out_shape=jax.ShapeDtypeStruct((N, N2), jnp.float32),
    )(Xa, Xb)


def _tc_gstep(Ps=None, si=None, x=None, add=None, W=None, pre_relu=False,
              post_relu=False, oscale=None, colsplit=False, pcat=False,
              mt=1000):
    N = Ps[0].shape[1] if Ps is not None else x.shape[0]
    M = N // mt

    if W is not None:
        Fo = W.shape[1]
    elif Ps is not None:
        Fo = sum(p.shape[2] * (2 if pcat else 1) for p in Ps)
    else:
        Fo = x.shape[1]
    CG = Fo // _F
    if colsplit and CG == 1:
        colsplit = False

    grid = (M, CG) if colsplit else (M,)
    if colsplit:
        def map_row(m, j):
            return (m, 0)
    else:
        def map_row(m):
            return (m, 0)

    if colsplit:
        def map_P(m, j):
            return (0, m, 0)
    else:
        def map_P(m):
            return (0, m, 0)

    arrays, specs = [], []
    if Ps is not None:
        for p in Ps:
            arrays.append(p)
            specs.append(pl.BlockSpec((2, mt, p.shape[2]), map_P))
        arrays.append(si)
        specs.append(pl.BlockSpec((mt, 1), map_row))
    else:
        arrays.append(x)
        specs.append(pl.BlockSpec((mt, x.shape[1]), map_row))
    if add is not None:
        arrays.append(add)
        specs.append(pl.BlockSpec((mt, add.shape[1]), map_row))
    if W is not None:
        Fk = W.shape[0]
        if colsplit:
            W3 = W.reshape(Fk, CG, _F).transpose(1, 0, 2)
            arrays.append(W3)
            specs.append(pl.BlockSpec((1, Fk, _F), lambda m, j: (j, 0, 0)))
        else:
            arrays.append(W)
            specs.append(pl.BlockSpec((Fk, Fo), lambda m: (0, 0)))
    if oscale is not None:
        arrays.append(oscale)
        specs.append(pl.BlockSpec((mt, 1), map_row))

    if colsplit:
        out_shape = jax.ShapeDtypeStruct((CG, N, _F), jnp.float32)
        out_spec = pl.BlockSpec((1, mt, _F), lambda m, j: (j, m, 0))
    else:
        out_shape = jax.ShapeDtypeStruct((N, Fo), jnp.float32)
        out_spec = pl.BlockSpec((mt, Fo), lambda m: (m, 0))

    def body(*refs):
        it = iter(refs)
        if Ps is not None:
            parts = []
            for _ in Ps:
                q = next(it)[...]
                if pcat:
                    parts.extend([q[0], q[1]])
                else:
                    parts.append(q[0] + q[1])
            u = parts[0] if len(parts) == 1 else jnp.concatenate(parts, axis=1)
            u = u * next(it)[...]
        else:
            u = next(it)[...]
        if pre_relu:
            u = jnp.maximum(u, 0.0)
        if add is not None:
            u = u + next(it)[...]
        if W is not None:
            w = next(it)[...]
            u = _bf16_dot(u, w[0] if colsplit else w)
        if post_relu:
            u = jnp.maximum(u, 0.0)
        if oscale is not None:
            u = u * next(it)[...]
        out_ref = next(it)
        out_ref[...] = u[None] if colsplit else u

    return pl.pallas_call(
        body,
        grid=grid,
        in_specs=specs,
        out_specs=out_spec,
        out_shape=out_shape,
    )(*arrays)


def kernel(x, B, edge_index, W_d1, W_d2, W_d3, W_g1, W_g2, W_g3, W_g4,
           W_s1, W_s2, W_s3, W_a1, W_a2, W_a3, W_a5):
    N, D = x.shape
    E = edge_index.shape[1]
    src = edge_index[0].astype(jnp.int32)
    dst = edge_index[1].astype(jnp.int32)

    hist = _make_deg(N, E)(src, dst)
    sc = _tc_scales(hist.reshape(2, _NC * _NS, N))
    so = sc[0].reshape(N, 1)
    si = sc[1].reshape(N, 1)

    srcg = src.reshape(_NC, _NS, -1, _K, _C)
    dstg = dst.reshape(_NC, _NS, -1, _K, _C)
    srcall = src.reshape(_NS, -1, _K, _C)
    dstall = dst.reshape(_NS, -1, _K, _C)
    src2f = jnp.stack([srcall, srcall + N])
    dst2f = jnp.stack([dstall, dstall])

    def agg(t):
        return _make_agg(N, E, N)(t, srcg, dstg)

    def agg256(t):
        return _make_agg(N, E, 2 * N, True)(t.reshape(2 * N, _F), src2f, dst2f)

    t1 = _tc_gstep(x=x, oscale=so)
    q1 = agg(t1)
    h1 = _tc_big_mm_relu(B, W_d1)
    enc1 = _tc_gstep(Ps=[q1], si=si, W=W_g1, post_relu=True)
    t2 = _tc_gstep(x=enc1, add=h1, W=W_g2, oscale=so, colsplit=True)
    q2 = agg256(t2)
    h2 = _tc_gstep(x=h1, W=W_d2, post_relu=True)
    t3 = _tc_gstep(Ps=[q2], pcat=True, si=si, pre_relu=True, add=h2, W=W_g3,
                   oscale=so)
    q3 = agg(t3)
    z_a = _tc_gstep(x=h2, W=W_d3, post_relu=True)
    t4 = _tc_gstep(Ps=[q3], si=si, pre_relu=True, add=z_a, W=W_g4, oscale=so)
    q4 = agg(t4)
    se1 = _tc_gstep(x=z_a, W=W_s1, post_relu=True)
    t5 = _tc_gstep(Ps=[q4], si=si, pre_relu=True, W=W_a1, oscale=so)
    q5 = agg(t5)
    se2 = _tc_gstep(x=se1, W=W_s2, post_relu=True)
    seu = _tc_big_out(se2, W_s3, trans_b=False)
    so_tap6 = so + 0.0 * seu[:1, :1]
    t6 = _tc_gstep(Ps=[q5], si=si, pre_relu=True, oscale=so_tap6)
    q6 = agg(t6)
    t7 = _tc_gstep(Ps=[q6], si=si, W=W_a2, post_relu=True, oscale=so,
                   colsplit=True)
    q7 = agg256(t7)
    a3 = _tc_gstep(Ps=[q7], pcat=True, si=si, W=W_a3, post_relu=True)
    t8 = _tc_gstep(x=a3, W=W_a5, oscale=so)
    q8 = agg(t8)
    z_st = _tc_big_out(a3, a3, trans_b=True)
    si_tap = si + 0.0 * z_st[:1, :1]
    a5 = _tc_gstep(Ps=[q8], si=si_tap, pre_relu=True)

    return (seu, a5, z_st)


# speedup vs baseline: 1.0174x; 1.0174x over previous
import functools

import jax
import jax.numpy as jnp
from jax import lax
from jax.experimental import pallas as pl
from jax.experimental.pallas import tpu as pltpu
from jax.experimental.pallas import tpu_sc as plsc

_NC = 2
_NS = 16
_C = 100
_K = 10
_F = 128


@functools.lru_cache(maxsize=None)
def _make_deg(N, E):
    NW = _NC * _NS
    EP = E // NW
    NR = N // 16
    assert E % NW == 0 and EP % 16 == 0 and N % 16 == 0
    mesh = plsc.VectorSubcoreMesh(core_axis_name="c", subcore_axis_name="s")

    @functools.partial(
        pl.kernel,
        out_type=jax.ShapeDtypeStruct((2, NW, NR, 16), jnp.float32),
        mesh=mesh,
        compiler_params=pltpu.CompilerParams(needs_layout_passes=False),
        scratch_types=[
            pltpu.VMEM((EP,), jnp.int32),
            pltpu.VMEM((NR, 16), jnp.float32),
        ],
    )
    def deg(srcv, dstv, out, idxv, hist):
        c = lax.axis_index("c")
        s = lax.axis_index("s")
        w = c * _NS + s
        ones16 = jnp.ones((16,), jnp.float32)

        U = 5
        assert NR % U == 0 and (EP // 16) % U == 0

        def zbody(i, carry):
            for k in range(U):
                hist[i * U + k, :] = jnp.zeros((16,), jnp.float32)
            return carry

        def abody(j, carry):
            for k in range(U):
                iv = idxv[pl.ds((j * U + k) * 16, 16)]
                plsc.addupdate_scatter(
                    hist,
                    [lax.shift_right_logical(iv, 4), lax.bitwise_and(iv, 15)],
                    ones16)
            return carry

        for row, ix_hbm in ((0, srcv), (1, dstv)):
            lax.fori_loop(0, NR // U, zbody, 0)
            pltpu.sync_copy(ix_hbm.at[pl.ds(w * EP, EP)], idxv)
            lax.fori_loop(0, EP // 16 // U, abody, 0)
            pltpu.sync_copy(hist, out.at[row, w])

    return deg


@functools.lru_cache(maxsize=None)
def _make_agg(N, E, T, full=False):
    EP = E // _NS if full else E // (_NC * _NS)
    NCH = EP // _C
    NG = NCH // _K
    assert EP % _C == 0 and NCH % _K == 0 and NG % 2 == 0 and _K % 2 == 0
    SR = 8 * (N // (8 * _NS))
    REM = N - SR * _NS
    ZR = 16
    assert SR % ZR == 0 and REM % ZR == 0
    mesh = plsc.VectorSubcoreMesh(core_axis_name="c", subcore_axis_name="s")

    @functools.partial(
        pl.kernel,
        out_type=jax.ShapeDtypeStruct((_NC, N, _F), jnp.float32),
        mesh=mesh,
        compiler_params=pltpu.CompilerParams(needs_layout_passes=False),
        scratch_types=[
            pltpu.VMEM((_K, _C), jnp.int32),
            pltpu.VMEM((_K, _C), jnp.int32),
            pltpu.VMEM((_K, _C), jnp.int32),
            pltpu.VMEM((_K, _C), jnp.int32),
            pltpu.VMEM((_C, _F), jnp.float32),
            pltpu.VMEM((_C, _F), jnp.float32),
            pltpu.VMEM((_C, _F), jnp.float32),
            pltpu.VMEM((ZR, _F), jnp.float32),
            pltpu.VMEM_SHARED((N, _F), jnp.float32),
            pltpu.SemaphoreType.DMA,
            pltpu.SemaphoreType.DMA,
            pltpu.SemaphoreType.DMA,
            pltpu.SemaphoreType.DMA,
            pltpu.SemaphoreType.DMA,
            pltpu.SemaphoreType.DMA,
            pltpu.SemaphoreType.DMA,
            pltpu.SemaphoreType.DMA,
        ],
    )
    def agg(tbl, srcix, dstix, out, s0, d0, s1, d1, rb0, rb1, rb2, zb, acc,
            is0, is1, gs0, gs1, gs2, ss0, ss1, ss2):
        c = lax.axis_index("c")
        s = lax.axis_index("s")

        def stage(g, sbuf, dbuf, sem):
            pltpu.async_copy(srcix.at[c, s, g], sbuf, sem)
            pltpu.async_copy(dstix.at[c, s, g], dbuf, sem)

        def stage_wait(g, sbuf, dbuf, sem):
            pltpu.make_async_copy(srcix.at[c, s, g], sbuf, sem).wait()
            pltpu.make_async_copy(dstix.at[c, s, g], dbuf, sem).wait()

        rb = (rb0, rb1, rb2)
        gs = (gs0, gs1, gs2)
        ss = (ss0, ss1, ss2)

        def zbody(i, carry):
            r = i // (_F // 16)
            col = (i % (_F // 16)) * 16
            zb[r, pl.ds(col, 16)] = jnp.zeros((16,), jnp.float32)
            return carry

        lax.fori_loop(0, ZR * (_F // 16), zbody, 0)

        zd = [pltpu.async_copy(zb, acc.at[pl.ds(s * SR + i * ZR, ZR)], is0)
              for i in range(SR // ZR)]
        if REM:
            @pl.when(s == 0)
            def _():
                for i in range(REM // ZR):
                    pltpu.sync_copy(zb, acc.at[pl.ds(_NS * SR + i * ZR, ZR)])
        for d in zd:
            d.wait()
        stage(0, s0, d0, is0)
        stage(1, s1, d1, is1)
        plsc.subcore_barrier()

        def gbody(gg, carry):
            g0 = 2 * gg
            g1 = g0 + 1

            def bufs(u):
                return (s0, d0, u) if u < _K else (s1, d1, u - _K)

            stage_wait(g0, s0, d0, is0)
            gd, sd = {}, {}
            for u in range(2 * _K):
                if u == _K:
                    stage_wait(g1, s1, d1, is1)
                b = u % 3
                if u >= 3:
                    sd[u - 3].wait()
                    if u - 3 == _K - 1:
                        @pl.when(gg < NG // 2 - 1)
                        def _():
                            stage(g0 + 2, s0, d0, is0)
                sb, db, r = bufs(u)
                gd[u] = pltpu.async_copy(tbl.at[sb.at[r]], rb[b], gs[b])
                if u >= 1:
                    _, pdb, pr = bufs(u - 1)
                    gd[u - 1].wait()
                    sd[u - 1] = pltpu.async_copy(
                        rb[(u - 1) % 3], acc.at[pdb.at[pr]], ss[(u - 1) % 3],
                        add=True)
            gd[2 * _K - 1].wait()
            _, pdb, pr = bufs(2 * _K - 1)
            sd[2 * _K - 1] = pltpu.async_copy(
                rb[(2 * _K - 1) % 3], acc.at[pdb.at[pr]], ss[(2 * _K - 1) % 3],
                add=True)
            for u in range(2 * _K - 3, 2 * _K):
                sd[u].wait()

            @pl.when(gg < NG // 2 - 1)
            def _():
                stage(g1 + 2, s1, d1, is1)

            return carry

        lax.fori_loop(0, NG // 2, gbody, 0)
        plsc.subcore_barrier()
        pltpu.sync_copy(acc.at[pl.ds(s * SR, SR)], out.at[c, pl.ds(s * SR, SR)])
        if REM:
            @pl.when(s == 0)
            def _():
                pltpu.sync_copy(acc.at[pl.ds(_NS * SR, REM)],
                                out.at[c, pl.ds(_NS * SR, REM)])

    return agg


def _bf16_dot(a, b, trans_b=False):
    dn = (((1,), (1,)), ((), ())) if trans_b else (((1,), (0,)), ((), ()))
    return lax.dot_general(a.astype(jnp.bfloat16), b.astype(jnp.bfloat16),
                           dn, preferred_element_type=jnp.float32)


def _tc_scales(hist):
    def body(h_ref, o_ref):
        d = jnp.sum(h_ref[...], axis=1)
        o_ref[...] = 1.0 / jnp.sqrt(jnp.maximum(d, 1.0))

    return pl.pallas_call(
        body,
        out_shape=jax.ShapeDtypeStruct((2, hist.shape[2]), jnp.float32),
    )(hist)


def _tc_big_mm_relu(Bm, W, mt=256):
    N, K = Bm.shape
    Fo = W.shape[1]
    M = pl.cdiv(N, mt)
    Wb = W.astype(jnp.bfloat16)

    def body(b_ref, w_ref, o_ref):
        o_ref[...] = jnp.maximum(_bf16_dot(b_ref[...], w_ref[...]), 0.0)

    return pl.pallas_call(
        body,
        grid=(M,),
        in_specs=[
            pl.BlockSpec((mt, K), lambda m: (m, 0)),
            pl.BlockSpec((K, Fo), lambda m: (0, 0)),
        ],
        out_specs=pl.BlockSpec((mt, Fo), lambda m: (m, 0)),
        out_shape=jax.ShapeDtypeStruct((N, Fo), jnp.float32),
    )(Bm, Wb)


def _tc_big_out(Xa, Xb, trans_b, mt=1000, nt=2560):
    N = Xa.shape[0]
    N2 = Xb.shape[0] if trans_b else Xb.shape[1]
    K = Xa.shape[1]
    M, NT = pl.cdiv(N, mt), pl.cdiv(N2, nt)

    def body(a_ref, b_ref, o_ref):
        o_ref[...] = jax.nn.sigmoid(_bf16_dot(a_ref[...], b_ref[...], trans_b))

    if trans_b:
        b_spec = pl.BlockSpec((nt, K), lambda m, j: (j, 0))
    else:
        b_spec = pl.BlockSpec((K, nt), lambda m, j: (0, j))
    return pl.pallas_call(
        body,
        grid=(M, NT),
        in_specs=[pl.BlockSpec((mt, K), lambda m, j: (m, 0)), b_spec],
        out_specs=pl.BlockSpec((mt, nt), lambda m, j: (m, j)),
        out_shape=jax.ShapeDtypeStruct((N, N2), jnp.float32),
    )(Xa, Xb)


def _tc_gstep(Ps=None, si=None, x=None, add=None, W=None, pre_relu=False,
              post_relu=False, oscale=None, colsplit=False, pcat=False,
              mt=1000):
    N = Ps[0].shape[1] if Ps is not None else x.shape[0]
    M = N // mt

    if W is not None:
        Fo = W.shape[1]
    elif Ps is not None:
        Fo = sum(p.shape[2] * (2 if pcat else 1) for p in Ps)
    else:
        Fo = x.shape[1]
    CG = Fo // _F
    if colsplit and CG == 1:
        colsplit = False

    grid = (M, CG) if colsplit else (M,)
    if colsplit:
        def map_row(m, j):
            return (m, 0)
    else:
        def map_row(m):
            return (m, 0)

    if colsplit:
        def map_P(m, j):
            return (0, m, 0)
    else:
        def map_P(m):
            return (0, m, 0)

    arrays, specs = [], []
    if Ps is not None:
        for p in Ps:
            arrays.append(p)
            specs.append(pl.BlockSpec((2, mt, p.shape[2]), map_P))
        arrays.append(si)
        specs.append(pl.BlockSpec((mt, 1), map_row))
    else:
        arrays.append(x)
        specs.append(pl.BlockSpec((mt, x.shape[1]), map_row))
    if add is not None:
        arrays.append(add)
        specs.append(pl.BlockSpec((mt, add.shape[1]), map_row))
    if W is not None:
        Fk = W.shape[0]
        if colsplit:
            W3 = W.reshape(Fk, CG, _F).transpose(1, 0, 2)
            arrays.append(W3)
            specs.append(pl.BlockSpec((1, Fk, _F), lambda m, j: (j, 0, 0)))
        else:
            arrays.append(W)
            specs.append(pl.BlockSpec((Fk, Fo), lambda m: (0, 0)))
    if oscale is not None:
        arrays.append(oscale)
        specs.append(pl.BlockSpec((mt, 1), map_row))

    if colsplit:
        out_shape = jax.ShapeDtypeStruct((CG, N, _F), jnp.float32)
        out_spec = pl.BlockSpec((1, mt, _F), lambda m, j: (j, m, 0))
    else:
        out_shape = jax.ShapeDtypeStruct((N, Fo), jnp.float32)
        out_spec = pl.BlockSpec((mt, Fo), lambda m: (m, 0))

    def body(*refs):
        it = iter(refs)
        if Ps is not None:
            parts = []
            for _ in Ps:
                q = next(it)[...]
                if pcat:
                    parts.extend([q[0], q[1]])
                else:
                    parts.append(q[0] + q[1])
            u = parts[0] if len(parts) == 1 else jnp.concatenate(parts, axis=1)
            u = u * next(it)[...]
        else:
            u = next(it)[...]
        if pre_relu:
            u = jnp.maximum(u, 0.0)
        if add is not None:
            u = u + next(it)[...]
        if W is not None:
            w = next(it)[...]
            u = _bf16_dot(u, w[0] if colsplit else w)
        if post_relu:
            u = jnp.maximum(u, 0.0)
        if oscale is not None:
            u = u * next(it)[...]
        out_ref = next(it)
        out_ref[...] = u[None] if colsplit else u

    return pl.pallas_call(
        body,
        grid=grid,
        in_specs=specs,
        out_specs=out_spec,
        out_shape=out_shape,
    )(*arrays)


def kernel(x, B, edge_index, W_d1, W_d2, W_d3, W_g1, W_g2, W_g3, W_g4,
           W_s1, W_s2, W_s3, W_a1, W_a2, W_a3, W_a5):
    N, D = x.shape
    E = edge_index.shape[1]
    src = edge_index[0].astype(jnp.int32)
    dst = edge_index[1].astype(jnp.int32)

    hist = _make_deg(N, E)(src, dst)
    sc = _tc_scales(hist.reshape(2, _NC * _NS, N))
    so = sc[0].reshape(N, 1)
    si = sc[1].reshape(N, 1)

    srcg = src.reshape(_NC, _NS, -1, _K, _C)
    dstg = dst.reshape(_NC, _NS, -1, _K, _C)
    srcall = src.reshape(_NS, -1, _K, _C)
    dstall = dst.reshape(_NS, -1, _K, _C)
    src2f = jnp.stack([srcall, srcall + N])
    dst2f = jnp.stack([dstall, dstall])

    def agg(t):
        return _make_agg(N, E, N)(t, srcg, dstg)

    def agg256(t):
        return _make_agg(N, E, 2 * N, True)(t.reshape(2 * N, _F), src2f, dst2f)

    t1 = _tc_gstep(x=x, oscale=so)
    q1 = agg(t1)
    h1 = _tc_big_mm_relu(B, W_d1)
    enc1 = _tc_gstep(Ps=[q1], si=si, W=W_g1, post_relu=True)
    t2 = _tc_gstep(x=enc1, add=h1, W=W_g2, oscale=so, colsplit=True)
    q2 = agg256(t2)
    h2 = _tc_gstep(x=h1, W=W_d2, post_relu=True)
    t3 = _tc_gstep(Ps=[q2], pcat=True, si=si, pre_relu=True, add=h2, W=W_g3,
                   oscale=so)
    q3 = agg(t3)
    z_a = _tc_gstep(x=h2, W=W_d3, post_relu=True)
    t4 = _tc_gstep(Ps=[q3], si=si, pre_relu=True, add=z_a, W=W_g4, oscale=so)
    q4 = agg(t4)
    se1 = _tc_gstep(x=z_a, W=W_s1, post_relu=True)
    t5 = _tc_gstep(Ps=[q4], si=si, pre_relu=True, W=W_a1, oscale=so)
    q5 = agg(t5)
    se2 = _tc_gstep(x=se1, W=W_s2, post_relu=True)
    t6 = _tc_gstep(Ps=[q5], si=si, pre_relu=True, oscale=so)
    q6 = agg(t6)
    seu = _tc_big_out(se2, W_s3, trans_b=False)
    t7 = _tc_gstep(Ps=[q6], si=si, W=W_a2, post_relu=True, oscale=so,
                   colsplit=True)
    q7 = agg256(t7)
    a3 = _tc_gstep(Ps=[q7], pcat=True, si=si, W=W_a3, post_relu=True)
    so_tap = so + 0.0 * seu[:1, :1]
    t8 = _tc_gstep(x=a3, W=W_a5, oscale=so_tap)
    q8 = agg(t8)
    z_st = _tc_big_out(a3, a3, trans_b=True)
    si_tap = si + 0.0 * z_st[:1, :1]
    a5 = _tc_gstep(Ps=[q8], si=si_tap, pre_relu=True)

    return (seu, a5, z_st)


# speedup vs baseline: 1.0207x; 1.0032x over previous
import functools

import jax
import jax.numpy as jnp
from jax import lax
from jax.experimental import pallas as pl
from jax.experimental.pallas import tpu as pltpu
from jax.experimental.pallas import tpu_sc as plsc

_NC = 2
_NS = 16
_C = 100
_K = 10
_F = 128


@functools.lru_cache(maxsize=None)
def _make_deg(N, E):
    NW = _NC * _NS
    EP = E // NW
    NR = N // 16
    assert E % NW == 0 and EP % 16 == 0 and N % 16 == 0
    mesh = plsc.VectorSubcoreMesh(core_axis_name="c", subcore_axis_name="s")

    @functools.partial(
        pl.kernel,
        out_type=jax.ShapeDtypeStruct((2, NW, NR, 16), jnp.float32),
        mesh=mesh,
        compiler_params=pltpu.CompilerParams(needs_layout_passes=False),
        scratch_types=[
            pltpu.VMEM((EP,), jnp.int32),
            pltpu.VMEM((NR, 16), jnp.float32),
        ],
    )
    def deg(srcv, dstv, out, idxv, hist):
        c = lax.axis_index("c")
        s = lax.axis_index("s")
        w = c * _NS + s
        ones16 = jnp.ones((16,), jnp.float32)

        U = 5
        assert NR % U == 0 and (EP // 16) % U == 0

        def zbody(i, carry):
            for k in range(U):
                hist[i * U + k, :] = jnp.zeros((16,), jnp.float32)
            return carry

        def abody(j, carry):
            for k in range(U):
                iv = idxv[pl.ds((j * U + k) * 16, 16)]
                plsc.addupdate_scatter(
                    hist,
                    [lax.shift_right_logical(iv, 4), lax.bitwise_and(iv, 15)],
                    ones16)
            return carry

        for row, ix_hbm in ((0, srcv), (1, dstv)):
            lax.fori_loop(0, NR // U, zbody, 0)
            pltpu.sync_copy(ix_hbm.at[pl.ds(w * EP, EP)], idxv)
            lax.fori_loop(0, EP // 16 // U, abody, 0)
            pltpu.sync_copy(hist, out.at[row, w])

    return deg


@functools.lru_cache(maxsize=None)
def _make_agg(N, E, T, full=False):
    EP = E // _NS if full else E // (_NC * _NS)
    NCH = EP // _C
    NG = NCH // _K
    assert EP % _C == 0 and NCH % _K == 0 and NG % 2 == 0 and _K % 2 == 0
    SR = 8 * (N // (8 * _NS))
    REM = N - SR * _NS
    ZR = 16
    assert SR % ZR == 0 and REM % ZR == 0
    mesh = plsc.VectorSubcoreMesh(core_axis_name="c", subcore_axis_name="s")

    @functools.partial(
        pl.kernel,
        out_type=jax.ShapeDtypeStruct((_NC, N, _F), jnp.float32),
        mesh=mesh,
        compiler_params=pltpu.CompilerParams(needs_layout_passes=False),
        scratch_types=[
            pltpu.VMEM((_K, _C), jnp.int32),
            pltpu.VMEM((_K, _C), jnp.int32),
            pltpu.VMEM((_K, _C), jnp.int32),
            pltpu.VMEM((_K, _C), jnp.int32),
            pltpu.VMEM((_C, _F), jnp.float32),
            pltpu.VMEM((_C, _F), jnp.float32),
            pltpu.VMEM((_C, _F), jnp.float32),
            pltpu.VMEM((ZR, _F), jnp.float32),
            pltpu.VMEM_SHARED((N, _F), jnp.float32),
            pltpu.SemaphoreType.DMA,
            pltpu.SemaphoreType.DMA,
            pltpu.SemaphoreType.DMA,
            pltpu.SemaphoreType.DMA,
            pltpu.SemaphoreType.DMA,
            pltpu.SemaphoreType.DMA,
            pltpu.SemaphoreType.DMA,
            pltpu.SemaphoreType.DMA,
        ],
    )
    def agg(tbl, srcix, dstix, out, s0, d0, s1, d1, rb0, rb1, rb2, zb, acc,
            is0, is1, gs0, gs1, gs2, ss0, ss1, ss2):
        c = lax.axis_index("c")
        s = lax.axis_index("s")

        def stage(g, sbuf, dbuf, sem):
            pltpu.async_copy(srcix.at[c, s, g], sbuf, sem)
            pltpu.async_copy(dstix.at[c, s, g], dbuf, sem)

        def stage_wait(g, sbuf, dbuf, sem):
            pltpu.make_async_copy(srcix.at[c, s, g], sbuf, sem).wait()
            pltpu.make_async_copy(dstix.at[c, s, g], dbuf, sem).wait()

        rb = (rb0, rb1, rb2)
        gs = (gs0, gs1, gs2)
        ss = (ss0, ss1, ss2)

        def zbody(i, carry):
            r = i // (_F // 16)
            col = (i % (_F // 16)) * 16
            zb[r, pl.ds(col, 16)] = jnp.zeros((16,), jnp.float32)
            return carry

        lax.fori_loop(0, ZR * (_F // 16), zbody, 0)

        zd = [pltpu.async_copy(zb, acc.at[pl.ds(s * SR + i * ZR, ZR)], is0)
              for i in range(SR // ZR)]
        if REM:
            @pl.when(s == 0)
            def _():
                for i in range(REM // ZR):
                    pltpu.sync_copy(zb, acc.at[pl.ds(_NS * SR + i * ZR, ZR)])
        for d in zd:
            d.wait()
        stage(0, s0, d0, is0)
        stage(1, s1, d1, is1)
        plsc.subcore_barrier()

        def gbody(gg, carry):
            g0 = 2 * gg
            g1 = g0 + 1

            def bufs(u):
                return (s0, d0, u) if u < _K else (s1, d1, u - _K)

            stage_wait(g0, s0, d0, is0)
            gd, sd = {}, {}
            for u in range(2 * _K):
                if u == _K:
                    stage_wait(g1, s1, d1, is1)
                b = u % 3
                if u >= 3:
                    sd[u - 3].wait()
                    if u - 3 == _K - 1:
                        @pl.when(gg < NG // 2 - 1)
                        def _():
                            stage(g0 + 2, s0, d0, is0)
                sb, db, r = bufs(u)
                gd[u] = pltpu.async_copy(tbl.at[sb.at[r]], rb[b], gs[b])
                if u >= 1:
                    _, pdb, pr = bufs(u - 1)
                    gd[u - 1].wait()
                    sd[u - 1] = pltpu.async_copy(
                        rb[(u - 1) % 3], acc.at[pdb.at[pr]], ss[(u - 1) % 3],
                        add=True)
            gd[2 * _K - 1].wait()
            _, pdb, pr = bufs(2 * _K - 1)
            sd[2 * _K - 1] = pltpu.async_copy(
                rb[(2 * _K - 1) % 3], acc.at[pdb.at[pr]], ss[(2 * _K - 1) % 3],
                add=True)
            for u in range(2 * _K - 3, 2 * _K):
                sd[u].wait()

            @pl.when(gg < NG // 2 - 1)
            def _():
                stage(g1 + 2, s1, d1, is1)

            return carry

        lax.fori_loop(0, NG // 2, gbody, 0)
        plsc.subcore_barrier()
        pltpu.sync_copy(acc.at[pl.ds(s * SR, SR)], out.at[c, pl.ds(s * SR, SR)])
        if REM:
            @pl.when(s == 0)
            def _():
                pltpu.sync_copy(acc.at[pl.ds(_NS * SR, REM)],
                                out.at[c, pl.ds(_NS * SR, REM)])

    return agg


def _bf16_dot(a, b, trans_b=False):
    dn = (((1,), (1,)), ((), ())) if trans_b else (((1,), (0,)), ((), ()))
    return lax.dot_general(a.astype(jnp.bfloat16), b.astype(jnp.bfloat16),
                           dn, preferred_element_type=jnp.float32)


def _tc_scales(hist):
    def body(h_ref, o_ref):
        d = jnp.sum(h_ref[...], axis=1)
        o_ref[...] = 1.0 / jnp.sqrt(jnp.maximum(d, 1.0))

    return pl.pallas_call(
        body,
        out_shape=jax.ShapeDtypeStruct((2, hist.shape[2]), jnp.float32),
    )(hist)


def _tc_big_mm_relu(Bm, W, mt=400):
    N, K = Bm.shape
    Fo = W.shape[1]
    M = pl.cdiv(N, mt)
    Wb = W.astype(jnp.bfloat16)

    def body(b_ref, w_ref, o_ref):
        o_ref[...] = jnp.maximum(_bf16_dot(b_ref[...], w_ref[...]), 0.0)

    return pl.pallas_call(
        body,
        grid=(M,),
        in_specs=[
            pl.BlockSpec((mt, K), lambda m: (m, 0)),
            pl.BlockSpec((K, Fo), lambda m: (0, 0)),
        ],
        out_specs=pl.BlockSpec((mt, Fo), lambda m: (m, 0)),
        out_shape=jax.ShapeDtypeStruct((N, Fo), jnp.float32),
    )(Bm, Wb)


def _tc_big_out(Xa, Xb, trans_b, mt=1000, nt=2560):
    N = Xa.shape[0]
    N2 = Xb.shape[0] if trans_b else Xb.shape[1]
    K = Xa.shape[1]
    M, NT = pl.cdiv(N, mt), pl.cdiv(N2, nt)

    def body(a_ref, b_ref, o_ref):
        o_ref[...] = jax.nn.sigmoid(_bf16_dot(a_ref[...], b_ref[...], trans_b))

    if trans_b:
        b_spec = pl.BlockSpec((nt, K), lambda m, j: (j, 0))
    else:
        b_spec = pl.BlockSpec((K, nt), lambda m, j: (0, j))
    return pl.pallas_call(
        body,
        grid=(M, NT),
        in_specs=[pl.BlockSpec((mt, K), lambda m, j: (m, 0)), b_spec],
        out_specs=pl.BlockSpec((mt, nt), lambda m, j: (m, j)),
        out_shape=jax.ShapeDtypeStruct((N, N2), jnp.float32),
    )(Xa, Xb)


def _tc_gstep(Ps=None, si=None, x=None, add=None, W=None, pre_relu=False,
              post_relu=False, oscale=None, colsplit=False, pcat=False,
              mt=1000):
    N = Ps[0].shape[1] if Ps is not None else x.shape[0]
    M = N // mt

    if W is not None:
        Fo = W.shape[1]
    elif Ps is not None:
        Fo = sum(p.shape[2] * (2 if pcat else 1) for p in Ps)
    else:
        Fo = x.shape[1]
    CG = Fo // _F
    if colsplit and CG == 1:
        colsplit = False

    grid = (M, CG) if colsplit else (M,)
    if colsplit:
        def map_row(m, j):
            return (m, 0)
    else:
        def map_row(m):
            return (m, 0)

    if colsplit:
        def map_P(m, j):
            return (0, m, 0)
    else:
        def map_P(m):
            return (0, m, 0)

    arrays, specs = [], []
    if Ps is not None:
        for p in Ps:
            arrays.append(p)
            specs.append(pl.BlockSpec((2, mt, p.shape[2]), map_P))
        arrays.append(si)
        specs.append(pl.BlockSpec((mt, 1), map_row))
    else:
        arrays.append(x)
        specs.append(pl.BlockSpec((mt, x.shape[1]), map_row))
    if add is not None:
        arrays.append(add)
        specs.append(pl.BlockSpec((mt, add.shape[1]), map_row))
    if W is not None:
        Fk = W.shape[0]
        if colsplit:
            W3 = W.reshape(Fk, CG, _F).transpose(1, 0, 2)
            arrays.append(W3)
            specs.append(pl.BlockSpec((1, Fk, _F), lambda m, j: (j, 0, 0)))
        else:
            arrays.append(W)
            specs.append(pl.BlockSpec((Fk, Fo), lambda m: (0, 0)))
    if oscale is not None:
        arrays.append(oscale)
        specs.append(pl.BlockSpec((mt, 1), map_row))

    if colsplit:
        out_shape = jax.ShapeDtypeStruct((CG, N, _F), jnp.float32)
        out_spec = pl.BlockSpec((1, mt, _F), lambda m, j: (j, m, 0))
    else:
        out_shape = jax.ShapeDtypeStruct((N, Fo), jnp.float32)
        out_spec = pl.BlockSpec((mt, Fo), lambda m: (m, 0))

    def body(*refs):
        it = iter(refs)
        if Ps is not None:
            parts = []
            for _ in Ps:
                q = next(it)[...]
                if pcat:
                    parts.extend([q[0], q[1]])
                else:
                    parts.append(q[0] + q[1])
            u = parts[0] if len(parts) == 1 else jnp.concatenate(parts, axis=1)
            u = u * next(it)[...]
        else:
            u = next(it)[...]
        if pre_relu:
            u = jnp.maximum(u, 0.0)
        if add is not None:
            u = u + next(it)[...]
        if W is not None:
            w = next(it)[...]
            u = _bf16_dot(u, w[0] if colsplit else w)
        if post_relu:
            u = jnp.maximum(u, 0.0)
        if oscale is not None:
            u = u * next(it)[...]
        out_ref = next(it)
        out_ref[...] = u[None] if colsplit else u

    return pl.pallas_call(
        body,
        grid=grid,
        in_specs=specs,
        out_specs=out_spec,
        out_shape=out_shape,
    )(*arrays)


def kernel(x, B, edge_index, W_d1, W_d2, W_d3, W_g1, W_g2, W_g3, W_g4,
           W_s1, W_s2, W_s3, W_a1, W_a2, W_a3, W_a5):
    N, D = x.shape
    E = edge_index.shape[1]
    src = edge_index[0].astype(jnp.int32)
    dst = edge_index[1].astype(jnp.int32)

    hist = _make_deg(N, E)(src, dst)
    sc = _tc_scales(hist.reshape(2, _NC * _NS, N))
    so = sc[0].reshape(N, 1)
    si = sc[1].reshape(N, 1)

    srcg = src.reshape(_NC, _NS, -1, _K, _C)
    dstg = dst.reshape(_NC, _NS, -1, _K, _C)
    srcall = src.reshape(_NS, -1, _K, _C)
    dstall = dst.reshape(_NS, -1, _K, _C)
    src2f = jnp.stack([srcall, srcall + N])
    dst2f = jnp.stack([dstall, dstall])

    def agg(t):
        return _make_agg(N, E, N)(t, srcg, dstg)

    def agg256(t):
        return _make_agg(N, E, 2 * N, True)(t.reshape(2 * N, _F), src2f, dst2f)

    t1 = _tc_gstep(x=x, oscale=so)
    q1 = agg(t1)
    h1 = _tc_big_mm_relu(B, W_d1)
    enc1 = _tc_gstep(Ps=[q1], si=si, W=W_g1, post_relu=True)
    t2 = _tc_gstep(x=enc1, add=h1, W=W_g2, oscale=so, colsplit=True)
    q2 = agg256(t2)
    h2 = _tc_gstep(x=h1, W=W_d2, post_relu=True)
    t3 = _tc_gstep(Ps=[q2], pcat=True, si=si, pre_relu=True, add=h2, W=W_g3,
                   oscale=so)
    q3 = agg(t3)
    z_a = _tc_gstep(x=h2, W=W_d3, post_relu=True)
    t4 = _tc_gstep(Ps=[q3], si=si, pre_relu=True, add=z_a, W=W_g4, oscale=so)
    q4 = agg(t4)
    se1 = _tc_gstep(x=z_a, W=W_s1, post_relu=True)
    t5 = _tc_gstep(Ps=[q4], si=si, pre_relu=True, W=W_a1, oscale=so)
    q5 = agg(t5)
    se2 = _tc_gstep(x=se1, W=W_s2, post_relu=True)
    t6 = _tc_gstep(Ps=[q5], si=si, pre_relu=True, oscale=so)
    q6 = agg(t6)
    seu = _tc_big_out(se2, W_s3, trans_b=False)
    t7 = _tc_gstep(Ps=[q6], si=si, W=W_a2, post_relu=True, oscale=so,
                   colsplit=True)
    q7 = agg256(t7)
    a3 = _tc_gstep(Ps=[q7], pcat=True, si=si, W=W_a3, post_relu=True)
    so_tap = so + 0.0 * seu[:1, :1]
    t8 = _tc_gstep(x=a3, W=W_a5, oscale=so_tap)
    q8 = agg(t8)
    z_st = _tc_big_out(a3, a3, trans_b=True)
    si_tap = si + 0.0 * z_st[:1, :1]
    a5 = _tc_gstep(Ps=[q8], si=si_tap, pre_relu=True)

    return (seu, a5, z_st)
